# baseline (speedup 1.0000x reference)
import jax
import jax.numpy as jnp
from jax import lax
from jax.experimental import pallas as pl
from jax.experimental.pallas import tpu as pltpu
from jax.experimental.pallas import tpu_sc as plsc

B = 1024
L = 20
C = 1000
FLAT = B * C * L
NC, NS = 2, 16
BPS = B // NS
LPC = L // NC
NSEG = 10
SEG = LPC // NSEG * C * B
SSEG = SEG // NS
NZD = 1
ZCH = SSEG // NZD


def _sc_body(t_hbm, z_hbm, out_hbm, t_v, offs_v, ones_v, zslab,
             sem_z0, sem_z1, sem_s):
    core = lax.axis_index("c")
    sub = lax.axis_index("s")
    zsems = (sem_z0, sem_z1)

    pltpu.sync_copy(z_hbm, zslab)

    def fire_seg(k):
        base = core * (NSEG * SEG) + k * SEG + sub * SSEG
        return [
            pltpu.async_copy(
                zslab, out_hbm.at[pl.ds(base + j * ZCH, ZCH)], zsems[k % 2]
            )
            for j in range(NZD)
        ]

    pending = {0: fire_seg(0), 1: fire_seg(1)}

    pltpu.sync_copy(t_hbm.at[pl.ds(sub * (BPS * L), BPS * L)], t_v)
    for j in range(4):
        ones_v[pl.ds(j * 16, 16)] = jnp.ones((16,), jnp.float32)

    lane = lax.iota(jnp.int32, 16)
    for lr in range(LPC):
        l_abs = core * LPC + lr
        for k4 in range(4):
            gidx = (k4 * 16 + lane) * L + l_abs
            vals = plsc.load_gather(t_v, [gidx])
            off = (
                l_abs * (C * B)
                + (sub >> 1) * 1024
                + (sub & 1) * 64
                + k4 * 16
                + (vals >> 3) * 8192
                + (vals & 7) * 128
                + lane
            )
            offs_v[lr, pl.ds(k4 * 16, 16)] = off

    scat = []
    for k in range(NSEG):
        for cp in pending.pop(k):
            cp.wait()
        plsc.subcore_barrier()
        scat.append(
            pltpu.async_copy(ones_v, out_hbm.at[offs_v.at[k]], sem_s)
        )
        if k + 2 < NSEG:
            pending[k + 2] = fire_seg(k + 2)
    for cp in scat:
        cp.wait()


@jax.jit
def _one_hot_sc(t_flat, zeros_src):
    mesh = plsc.VectorSubcoreMesh(core_axis_name="c", subcore_axis_name="s")
    run = pl.kernel(
        _sc_body,
        out_type=jax.ShapeDtypeStruct((FLAT,), jnp.float32),
        mesh=mesh,
        scratch_types=[
            pltpu.VMEM((BPS * L,), jnp.int32),
            pltpu.VMEM((NSEG, 64), jnp.int32),
            pltpu.VMEM((64,), jnp.float32),
            pltpu.VMEM((ZCH,), jnp.float32),
            pltpu.SemaphoreType.DMA,
            pltpu.SemaphoreType.DMA,
            pltpu.SemaphoreType.DMA,
        ],
        compiler_params=pltpu.CompilerParams(needs_layout_passes=False),
        name="one_hot_sc",
    )
    flat = run(t_flat, zeros_src)
    return (
        flat.reshape(L, C // 8, B // 128, 8, 128)
        .transpose(2, 4, 1, 3, 0)
        .reshape(B, C, L)
    )


def kernel(t, ones):
    del ones
    t_flat = t.reshape(-1).astype(jnp.int32)
    zeros_src = jnp.zeros((ZCH,), jnp.float32)
    return _one_hot_sc(t_flat, zeros_src)


# speedup vs baseline: 1.0564x; 1.0564x over previous
import jax
import jax.numpy as jnp
from jax import lax
from jax.experimental import pallas as pl
from jax.experimental.pallas import tpu as pltpu
from jax.experimental.pallas import tpu_sc as plsc

B = 1024
L = 20
C = 1000
FLAT = B * C * L
NC, NS = 2, 16
BPS = B // NS
LPC = L // NC
NSEG = 10
SEG = LPC // NSEG * C * B
SSEG = SEG // NS
NZD = 2
ZCH = SSEG // NZD


def _sc_body(t_hbm, z_hbm, out_hbm, t_v, offs_v, ones_v, zslab,
             sem_z0, sem_z1, sem_z2, sem_s):
    core = lax.axis_index("c")
    sub = lax.axis_index("s")
    zsems = (sem_z0, sem_z1, sem_z2)

    pltpu.sync_copy(z_hbm, zslab)

    def fire_seg(k):
        base = core * (NSEG * SEG) + k * SEG + sub * SSEG
        return [
            pltpu.async_copy(
                zslab, out_hbm.at[pl.ds(base + j * ZCH, ZCH)], zsems[k % 3]
            )
            for j in range(NZD)
        ]

    pending = {0: fire_seg(0), 1: fire_seg(1), 2: fire_seg(2)}

    pltpu.sync_copy(t_hbm.at[pl.ds(sub * (BPS * L), BPS * L)], t_v)
    for j in range(4):
        ones_v[pl.ds(j * 16, 16)] = jnp.ones((16,), jnp.float32)

    lane = lax.iota(jnp.int32, 16)
    for lr in range(LPC):
        l_abs = core * LPC + lr
        for k4 in range(4):
            gidx = (k4 * 16 + lane) * L + l_abs
            vals = plsc.load_gather(t_v, [gidx])
            off = (
                l_abs * (C * B)
                + (sub >> 1) * 1024
                + (sub & 1) * 64
                + k4 * 16
                + (vals >> 3) * 8192
                + (vals & 7) * 128
                + lane
            )
            offs_v[lr, pl.ds(k4 * 16, 16)] = off

    scat = []
    for k in range(NSEG):
        for cp in pending.pop(k):
            cp.wait()
        plsc.subcore_barrier()
        scat.append(
            pltpu.async_copy(ones_v, out_hbm.at[offs_v.at[k]], sem_s)
        )
        if k + 3 < NSEG:
            pending[k + 3] = fire_seg(k + 3)
    for cp in scat:
        cp.wait()


@jax.jit
def _one_hot_sc(t_flat, zeros_src):
    mesh = plsc.VectorSubcoreMesh(core_axis_name="c", subcore_axis_name="s")
    run = pl.kernel(
        _sc_body,
        out_type=jax.ShapeDtypeStruct((FLAT,), jnp.float32),
        mesh=mesh,
        scratch_types=[
            pltpu.VMEM((BPS * L,), jnp.int32),
            pltpu.VMEM((NSEG, 64), jnp.int32),
            pltpu.VMEM((64,), jnp.float32),
            pltpu.VMEM((ZCH,), jnp.float32),
            pltpu.SemaphoreType.DMA,
            pltpu.SemaphoreType.DMA,
            pltpu.SemaphoreType.DMA,
            pltpu.SemaphoreType.DMA,
        ],
        compiler_params=pltpu.CompilerParams(needs_layout_passes=False),
        name="one_hot_sc",
    )
    flat = run(t_flat, zeros_src)
    return (
        flat.reshape(L, C // 8, B // 128, 8, 128)
        .transpose(2, 4, 1, 3, 0)
        .reshape(B, C, L)
    )


def kernel(t, ones):
    del ones
    t_flat = t.reshape(-1).astype(jnp.int32)
    zeros_src = jnp.zeros((ZCH,), jnp.float32)
    return _one_hot_sc(t_flat, zeros_src)
